# Initial kernel scaffold; baseline (speedup 1.0000x reference)
import jax
import jax.numpy as jnp
from jax.experimental import pallas as pl


def kernel(x, edge_index, edge_type, W, b):
    raise NotImplementedError("write your pallas kernel here")


# speedup vs baseline: 2.6975x; 2.6975x over previous
import jax
import jax.numpy as jnp
from jax import lax
from jax.experimental import pallas as pl
from jax.experimental.pallas import tpu as pltpu
from jax.experimental.pallas import tpu_sc as plsc

N = 10000
E = 160000
D = 256
R = 7
OUT = 256
H = 128
NC = 2
NS = 16
LANES = 16
RN = R * N

CH = 128
EPT = 10240
E_PAD = EPT * NS
CPT = EPT // CH
ACC_ROWS = 10240
RPT = ACC_ROWS // NS
FB = 64

BN = 2000
NB = N // BN


def _tc_body(x_ref, w_ref, y_ref):
    y_ref[...] = lax.dot_general(
        x_ref[...], w_ref[...],
        (((1,), (1,)), ((), ())),
        preferred_element_type=jnp.float32)


def _tc_transform(x, W):
    return pl.pallas_call(
        _tc_body,
        grid=(NB, NC, R),
        in_specs=[
            pl.BlockSpec((BN, D), lambda nb, h, r: (nb, 0)),
            pl.BlockSpec((H, D), lambda nb, h, r: (h, r)),
        ],
        out_specs=pl.BlockSpec(
            (BN, H), lambda nb, h, r: (h * (R * NB) + r * NB + nb, 0)),
        out_shape=jax.ShapeDtypeStruct((NC * RN, H), jnp.float32),
    )(x, W)


def _sc_body(y_h, src_h, et_h, dst_h, b_h, out_h,
             src_c, et_c, g_c, dst_c, rows_v, obuf, b_v, acc, sem):
    cid = lax.axis_index("c")
    sid = lax.axis_index("s")

    zero16 = jnp.zeros((LANES,), jnp.float32)

    def zrow(i, c):
        for j in range(H // LANES):
            obuf[i, pl.ds(j * LANES, LANES)] = zero16
        return c

    lax.fori_loop(0, FB, zrow, 0)

    def zcp(k, c):
        pltpu.sync_copy(obuf.at[pl.ds(0, FB)],
                        acc.at[pl.ds(sid * RPT + k * FB, FB)])
        return c

    lax.fori_loop(0, RPT // FB, zcp, 0)

    plsc.subcore_barrier()

    half_off = lax.broadcast(cid * RN, (LANES,))
    ebase = sid * EPT

    def chunk(ch, c):
        base = ebase + ch * CH
        pltpu.sync_copy(src_h.at[pl.ds(base, CH)], src_c)
        pltpu.sync_copy(et_h.at[pl.ds(base, CH)], et_c)
        pltpu.sync_copy(dst_h.at[pl.ds(sid * CPT + ch, 1)], dst_c)
        for j in range(CH // LANES):
            sl = pl.ds(j * LANES, LANES)
            g_c[sl] = et_c[sl] * N + src_c[sl] + half_off
        pltpu.async_copy(y_h.at[g_c], rows_v, sem).wait()
        pltpu.sync_copy(rows_v, acc.at[dst_c.at[0]], add=True)
        return c

    lax.fori_loop(0, CPT, chunk, 0)

    plsc.subcore_barrier()

    pltpu.sync_copy(b_h.at[pl.ds(cid * H, H)], b_v)

    def flush(row0, nrows):
        pltpu.sync_copy(acc.at[pl.ds(row0, nrows)], obuf.at[pl.ds(0, nrows)])

        def rrow(i, c):
            for j in range(H // LANES):
                sl = pl.ds(j * LANES, LANES)
                obuf[i, sl] = jnp.maximum(obuf[i, sl] + b_v[sl], 0.0)
            return c

        lax.fori_loop(0, nrows, rrow, 0)
        pltpu.sync_copy(obuf.at[pl.ds(0, nrows)],
                        out_h.at[pl.ds(row0, nrows), pl.ds(cid * H, H)])

    @pl.when(sid < NS - 1)
    def _():
        def fblk(k, c):
            flush(sid * RPT + k * FB, FB)
            return c
        lax.fori_loop(0, RPT // FB, fblk, 0)

    @pl.when(sid == NS - 1)
    def _():
        tail0 = (NS - 1) * RPT

        def fblk(k, c):
            flush(tail0 + k * FB, FB)
            return c
        lax.fori_loop(0, (N - tail0) // FB, fblk, 0)
        flush(tail0 + ((N - tail0) // FB) * FB, N - tail0 - ((N - tail0) // FB) * FB)


def _sc_aggregate(y, src, et, dst2, b):
    mesh = plsc.VectorSubcoreMesh(
        core_axis_name="c", subcore_axis_name="s",
        num_cores=NC, num_subcores=NS)
    f = pl.kernel(
        _sc_body,
        out_type=jax.ShapeDtypeStruct((N, OUT), jnp.float32),
        mesh=mesh,
        scratch_types=[
            pltpu.VMEM((CH,), jnp.int32),
            pltpu.VMEM((CH,), jnp.int32),
            pltpu.VMEM((CH,), jnp.int32),
            pltpu.VMEM((1, CH), jnp.int32),
            pltpu.VMEM((CH, H), jnp.float32),
            pltpu.VMEM((FB, H), jnp.float32),
            pltpu.VMEM((H,), jnp.float32),
            pltpu.VMEM_SHARED((ACC_ROWS, H), jnp.float32),
            pltpu.SemaphoreType.DMA,
        ],
    )
    return f(y, src, et, dst2, b)


def kernel(x, edge_index, edge_type, W, b):
    src = edge_index[0].astype(jnp.int32)
    dst = edge_index[1].astype(jnp.int32)
    et = edge_type.astype(jnp.int32)
    pad = E_PAD - E
    src = jnp.concatenate([src, jnp.zeros((pad,), jnp.int32)])
    et = jnp.concatenate([et, jnp.zeros((pad,), jnp.int32)])
    dst = jnp.concatenate([dst, jnp.full((pad,), N, jnp.int32)])
    dst2 = dst.reshape(E_PAD // CH, CH)
    y = _tc_transform(x, W)
    return _sc_aggregate(y, src, et, dst2, b)


# speedup vs baseline: 3.5779x; 1.3264x over previous
import jax
import jax.numpy as jnp
from jax import lax
from jax.experimental import pallas as pl
from jax.experimental.pallas import tpu as pltpu
from jax.experimental.pallas import tpu_sc as plsc

N = 10000
E = 160000
D = 256
R = 7
OUT = 256
H = 128
NC = 2
NS = 16
LANES = 16
RN = R * N

CH = 128
EPT = 10240
E_PAD = EPT * NS
CPT = EPT // CH
ACC_ROWS = 10240
RPT = ACC_ROWS // NS
FB = 64

BN = 2000
NB = N // BN


def _tc_body(x_ref, w_ref, y_ref):
    y_ref[...] = lax.dot_general(
        x_ref[...], w_ref[...],
        (((1,), (1,)), ((), ())),
        preferred_element_type=jnp.float32)


def _tc_transform(x, W):
    return pl.pallas_call(
        _tc_body,
        grid=(NB, NC, R),
        in_specs=[
            pl.BlockSpec((BN, D), lambda nb, h, r: (nb, 0)),
            pl.BlockSpec((H, D), lambda nb, h, r: (h, r)),
        ],
        out_specs=pl.BlockSpec(
            (BN, H), lambda nb, h, r: (h * (R * NB) + r * NB + nb, 0)),
        out_shape=jax.ShapeDtypeStruct((NC * RN, H), jnp.float32),
    )(x, W)


def _sc_body(y_h, ip_h, b_h, out_h,
             idx0, idx1, g0, g1, db0, db1, rows0, rows1, obuf, b_v, acc,
             si0, si1, sg0, sg1, ss0, ss1):
    idxs = (idx0, idx1)
    gs = (g0, g1)
    dbs = (db0, db1)
    rows = (rows0, rows1)
    sis = (si0, si1)
    sgs = (sg0, sg1)
    sss = (ss0, ss1)
    cid = lax.axis_index("c")
    sid = lax.axis_index("s")

    zero16 = jnp.zeros((LANES,), jnp.float32)

    def zrow(i, c):
        for j in range(H // LANES):
            obuf[i, pl.ds(j * LANES, LANES)] = zero16
        return c

    lax.fori_loop(0, FB, zrow, 0)

    def zcp(k, c):
        pltpu.sync_copy(obuf.at[pl.ds(0, FB)],
                        acc.at[pl.ds(sid * RPT + k * FB, FB)])
        return c

    lax.fori_loop(0, RPT // FB, zcp, 0)

    plsc.subcore_barrier()

    half_off = lax.broadcast(cid * RN, (LANES,))
    cbase = sid * CPT

    def issue_idx(j, p):
        pltpu.async_copy(ip_h.at[cbase + j], idxs[p], sis[p])

    def wait_idx(j, p):
        pltpu.make_async_copy(ip_h.at[cbase + j], idxs[p], sis[p]).wait()

    def compute(p):
        for jj in range(CH // LANES):
            sl = pl.ds(jj * LANES, LANES)
            gs[p][sl] = idxs[p][1, sl] * N + idxs[p][0, sl] + half_off
            dbs[p][sl] = idxs[p][2, sl]

    def issue_gather(p):
        pltpu.async_copy(y_h.at[gs[p]], rows[p], sgs[p])

    def wait_gather(p):
        pltpu.make_async_copy(y_h.at[gs[p]], rows[p], sgs[p]).wait()

    def issue_scatter(p):
        pltpu.async_copy(rows[p], acc.at[dbs[p]], sss[p], add=True)

    def wait_scatter(p):
        pltpu.make_async_copy(rows[p], acc.at[dbs[p]], sss[p]).wait()

    issue_idx(0, 0)
    issue_idx(1, 1)
    wait_idx(0, 0)
    compute(0)
    issue_gather(0)

    def pair(k, c):
        for b in range(2):
            j = 2 * k + b
            p = b
            q = 1 - b

            @pl.when(j + 1 < CPT)
            def _():
                wait_idx(j + 1, q)

            @pl.when(j > 0)
            def _():
                wait_scatter(q)

            @pl.when(j + 1 < CPT)
            def _():
                compute(q)
                issue_gather(q)

            @pl.when(j + 2 < CPT)
            def _():
                issue_idx(j + 2, p)

            wait_gather(p)
            issue_scatter(p)
        return c

    lax.fori_loop(0, CPT // 2, pair, 0)
    wait_scatter((CPT - 1) % 2)

    plsc.subcore_barrier()

    pltpu.sync_copy(b_h.at[pl.ds(cid * H, H)], b_v)

    def flush(row0, nrows):
        pltpu.sync_copy(acc.at[pl.ds(row0, nrows)], obuf.at[pl.ds(0, nrows)])

        def rrow(i, c):
            for j in range(H // LANES):
                sl = pl.ds(j * LANES, LANES)
                obuf[i, sl] = jnp.maximum(obuf[i, sl] + b_v[sl], 0.0)
            return c

        lax.fori_loop(0, nrows, rrow, 0)
        pltpu.sync_copy(obuf.at[pl.ds(0, nrows)],
                        out_h.at[pl.ds(row0, nrows), pl.ds(cid * H, H)])

    @pl.when(sid < NS - 1)
    def _():
        def fblk(k, c):
            flush(sid * RPT + k * FB, FB)
            return c
        lax.fori_loop(0, RPT // FB, fblk, 0)

    @pl.when(sid == NS - 1)
    def _():
        tail0 = (NS - 1) * RPT

        def fblk(k, c):
            flush(tail0 + k * FB, FB)
            return c
        lax.fori_loop(0, (N - tail0) // FB, fblk, 0)
        flush(tail0 + ((N - tail0) // FB) * FB, N - tail0 - ((N - tail0) // FB) * FB)


def _sc_aggregate(y, ip, b):
    mesh = plsc.VectorSubcoreMesh(
        core_axis_name="c", subcore_axis_name="s",
        num_cores=NC, num_subcores=NS)
    f = pl.kernel(
        _sc_body,
        out_type=jax.ShapeDtypeStruct((N, OUT), jnp.float32),
        mesh=mesh,
        scratch_types=[
            pltpu.VMEM((3, CH), jnp.int32),
            pltpu.VMEM((3, CH), jnp.int32),
            pltpu.VMEM((CH,), jnp.int32),
            pltpu.VMEM((CH,), jnp.int32),
            pltpu.VMEM((CH,), jnp.int32),
            pltpu.VMEM((CH,), jnp.int32),
            pltpu.VMEM((CH, H), jnp.float32),
            pltpu.VMEM((CH, H), jnp.float32),
            pltpu.VMEM((FB, H), jnp.float32),
            pltpu.VMEM((H,), jnp.float32),
            pltpu.VMEM_SHARED((ACC_ROWS, H), jnp.float32),
            pltpu.SemaphoreType.DMA,
            pltpu.SemaphoreType.DMA,
            pltpu.SemaphoreType.DMA,
            pltpu.SemaphoreType.DMA,
            pltpu.SemaphoreType.DMA,
            pltpu.SemaphoreType.DMA,
        ],
    )
    return f(y, ip, b)


def kernel(x, edge_index, edge_type, W, b):
    src = edge_index[0].astype(jnp.int32)
    dst = edge_index[1].astype(jnp.int32)
    et = edge_type.astype(jnp.int32)
    pad = E_PAD - E
    src = jnp.concatenate([src, jnp.zeros((pad,), jnp.int32)])
    et = jnp.concatenate([et, jnp.zeros((pad,), jnp.int32)])
    dst = jnp.concatenate([dst, jnp.full((pad,), N, jnp.int32)])
    ip = jnp.stack([src.reshape(E_PAD // CH, CH),
                    et.reshape(E_PAD // CH, CH),
                    dst.reshape(E_PAD // CH, CH)], axis=1)
    y = _tc_transform(x, W)
    return _sc_aggregate(y, ip, b)


# speedup vs baseline: 3.5812x; 1.0009x over previous
import jax
import jax.numpy as jnp
from jax import lax
from jax.experimental import pallas as pl
from jax.experimental.pallas import tpu as pltpu
from jax.experimental.pallas import tpu_sc as plsc

N = 10000
E = 160000
D = 256
R = 7
OUT = 256
H = 128
NC = 2
NS = 16
LANES = 16
RN = R * N

CH = 128
EPT = 10240
E_PAD = EPT * NS
CPT = EPT // CH
ACC_ROWS = 10240
RPT = ACC_ROWS // NS
FB = 64

BN = 2000
NB = N // BN


def _tc_body(x_ref, w_ref, y_ref):
    y_ref[...] = lax.dot_general(
        x_ref[...], w_ref[...],
        (((1,), (1,)), ((), ())),
        preferred_element_type=jnp.float32)


def _tc_transform(x, W):
    return pl.pallas_call(
        _tc_body,
        grid=(NB, NC, R),
        in_specs=[
            pl.BlockSpec((BN, D), lambda nb, h, r: (nb, 0)),
            pl.BlockSpec((H, D), lambda nb, h, r: (h, r)),
        ],
        out_specs=pl.BlockSpec(
            (BN, H), lambda nb, h, r: (h * (R * NB) + r * NB + nb, 0)),
        out_shape=jax.ShapeDtypeStruct((NC * RN, H), jnp.float32),
    )(x, W)


def _sc_body(y_h, ip_h, b_h, out_h,
             idx0, idx1, g0, g1, db0, db1, rows0, rows1, obuf, b_v, acc,
             si0, si1, sg0, sg1, ss0, ss1):
    idxs = (idx0, idx1)
    gs = (g0, g1)
    dbs = (db0, db1)
    rows = (rows0, rows1)
    sis = (si0, si1)
    sgs = (sg0, sg1)
    sss = (ss0, ss1)
    cid = lax.axis_index("c")
    sid = lax.axis_index("s")

    with jax.named_scope("acc_zero"):
        zero16 = jnp.zeros((LANES,), jnp.float32)

        def zrow(i, c):
            for j in range(H // LANES):
                obuf[i, pl.ds(j * LANES, LANES)] = zero16
            return c

        lax.fori_loop(0, FB, zrow, 0)

        def zcp(k, c):
            pltpu.sync_copy(obuf.at[pl.ds(0, FB)],
                            acc.at[pl.ds(sid * RPT + k * FB, FB)])
            return c

        lax.fori_loop(0, RPT // FB, zcp, 0)

        plsc.subcore_barrier()

    half_off = lax.broadcast(cid * RN, (LANES,))
    cbase = sid * CPT

    def issue_idx(j, p):
        pltpu.async_copy(ip_h.at[cbase + j], idxs[p], sis[p])

    def wait_idx(j, p):
        pltpu.make_async_copy(ip_h.at[cbase + j], idxs[p], sis[p]).wait()

    def compute(p):
        for jj in range(CH // LANES):
            sl = pl.ds(jj * LANES, LANES)
            gs[p][sl] = idxs[p][1, sl] * N + idxs[p][0, sl] + half_off
            dbs[p][sl] = idxs[p][2, sl]

    def issue_gather(p):
        pltpu.async_copy(y_h.at[gs[p]], rows[p], sgs[p])

    def wait_gather(p):
        pltpu.make_async_copy(y_h.at[gs[p]], rows[p], sgs[p]).wait()

    def issue_scatter(p):
        pltpu.async_copy(rows[p], acc.at[dbs[p]], sss[p], add=True)

    def wait_scatter(p):
        pltpu.make_async_copy(rows[p], acc.at[dbs[p]], sss[p]).wait()

    with jax.named_scope("edge_sweep"):
        issue_idx(0, 0)
        issue_idx(1, 1)
        wait_idx(0, 0)
        compute(0)
        issue_gather(0)

        def pair(k, c):
            for b in range(2):
                j = 2 * k + b
                p = b
                q = 1 - b

                @pl.when(j + 1 < CPT)
                def _():
                    wait_idx(j + 1, q)

                @pl.when(j > 0)
                def _():
                    wait_scatter(q)

                @pl.when(j + 1 < CPT)
                def _():
                    compute(q)
                    issue_gather(q)

                @pl.when(j + 2 < CPT)
                def _():
                    issue_idx(j + 2, p)

                wait_gather(p)
                issue_scatter(p)
            return c

        lax.fori_loop(0, CPT // 2, pair, 0)
        wait_scatter((CPT - 1) % 2)

        plsc.subcore_barrier()

    with jax.named_scope("bias_relu_out"):
        pltpu.sync_copy(b_h.at[pl.ds(cid * H, H)], b_v)

        def flush(row0, nrows):
            pltpu.sync_copy(acc.at[pl.ds(row0, nrows)], obuf.at[pl.ds(0, nrows)])

            def rrow(i, c):
                for j in range(H // LANES):
                    sl = pl.ds(j * LANES, LANES)
                    obuf[i, sl] = jnp.maximum(obuf[i, sl] + b_v[sl], 0.0)
                return c

            lax.fori_loop(0, nrows, rrow, 0)
            pltpu.sync_copy(obuf.at[pl.ds(0, nrows)],
                            out_h.at[pl.ds(row0, nrows), pl.ds(cid * H, H)])

        @pl.when(sid < NS - 1)
        def _():
            def fblk(k, c):
                flush(sid * RPT + k * FB, FB)
                return c
            lax.fori_loop(0, RPT // FB, fblk, 0)

        @pl.when(sid == NS - 1)
        def _():
            tail0 = (NS - 1) * RPT

            def fblk(k, c):
                flush(tail0 + k * FB, FB)
                return c
            lax.fori_loop(0, (N - tail0) // FB, fblk, 0)
            flush(tail0 + ((N - tail0) // FB) * FB,
                  N - tail0 - ((N - tail0) // FB) * FB)


def _sc_aggregate(y, ip, b):
    mesh = plsc.VectorSubcoreMesh(
        core_axis_name="c", subcore_axis_name="s",
        num_cores=NC, num_subcores=NS)
    f = pl.kernel(
        _sc_body,
        out_type=jax.ShapeDtypeStruct((N, OUT), jnp.float32),
        mesh=mesh,
        scratch_types=[
            pltpu.VMEM((3, CH), jnp.int32),
            pltpu.VMEM((3, CH), jnp.int32),
            pltpu.VMEM((CH,), jnp.int32),
            pltpu.VMEM((CH,), jnp.int32),
            pltpu.VMEM((CH,), jnp.int32),
            pltpu.VMEM((CH,), jnp.int32),
            pltpu.VMEM((CH, H), jnp.float32),
            pltpu.VMEM((CH, H), jnp.float32),
            pltpu.VMEM((FB, H), jnp.float32),
            pltpu.VMEM((H,), jnp.float32),
            pltpu.VMEM_SHARED((ACC_ROWS, H), jnp.float32),
            pltpu.SemaphoreType.DMA,
            pltpu.SemaphoreType.DMA,
            pltpu.SemaphoreType.DMA,
            pltpu.SemaphoreType.DMA,
            pltpu.SemaphoreType.DMA,
            pltpu.SemaphoreType.DMA,
        ],
    )
    return f(y, ip, b)


def kernel(x, edge_index, edge_type, W, b):
    src = edge_index[0].astype(jnp.int32)
    dst = edge_index[1].astype(jnp.int32)
    et = edge_type.astype(jnp.int32)
    pad = E_PAD - E
    src = jnp.concatenate([src, jnp.zeros((pad,), jnp.int32)])
    et = jnp.concatenate([et, jnp.zeros((pad,), jnp.int32)])
    dst = jnp.concatenate([dst, jnp.full((pad,), N, jnp.int32)])
    ip = jnp.stack([src.reshape(E_PAD // CH, CH),
                    et.reshape(E_PAD // CH, CH),
                    dst.reshape(E_PAD // CH, CH)], axis=1)
    y = _tc_transform(x, W)
    return _sc_aggregate(y, ip, b)
